# baseline (speedup 1.0000x reference)
import functools

import jax
import jax.numpy as jnp
from jax import lax
from jax.experimental import pallas as pl
from jax.experimental.pallas import tpu as pltpu
from jax.experimental.pallas import tpu_sc as plsc

NUM_SC = 2
NUM_SUBCORES = 16
LANES = 16


def kernel(labels, train, force_drop_ids, embedding_table):
    del train
    B = labels.shape[0]
    V, D = embedding_table.shape
    NW = NUM_SC * NUM_SUBCORES
    b_per_w = B // NW
    W = 64
    n_chunks = b_per_w // W

    labels32 = labels.astype(jnp.int32)
    drops32 = force_drop_ids.astype(jnp.int32)

    mesh = plsc.VectorSubcoreMesh(core_axis_name="c", subcore_axis_name="s")

    @functools.partial(
        pl.kernel,
        mesh=mesh,
        out_type=jax.ShapeDtypeStruct((B, D), jnp.float32),
        scratch_types=[
            pltpu.VMEM((b_per_w,), jnp.int32),
            pltpu.VMEM((b_per_w,), jnp.int32),
            pltpu.VMEM((W, D), jnp.float32),
        ],
    )
    def emb(table_hbm, lab_hbm, fdi_hbm, out_hbm, lab_v, fdi_v, rows_v):
        wid = lax.axis_index("s") * NUM_SC + lax.axis_index("c")
        base = wid * b_per_w

        pltpu.sync_copy(lab_hbm.at[pl.ds(base, b_per_w)], lab_v)
        pltpu.sync_copy(fdi_hbm.at[pl.ds(base, b_per_w)], fdi_v)

        @pl.loop(0, b_per_w, step=LANES)
        def _(i):
            sl = pl.ds(i, LANES)
            lab_v[sl] = jnp.where(fdi_v[sl] == 1, V - 1, lab_v[sl])

        @pl.loop(0, n_chunks)
        def _(c):
            pltpu.sync_copy(table_hbm.at[lab_v.at[pl.ds(c * W, W)]], rows_v)
            pltpu.sync_copy(rows_v, out_hbm.at[pl.ds(base + c * W, W)])

    return emb(embedding_table, labels32, drops32)
